# baseline (speedup 1.0000x reference)
import dataclasses
import functools

import jax
import jax.numpy as jnp
from jax import lax
from jax.experimental import pallas as pl
from jax.experimental.pallas import tpu as pltpu
from jax.experimental.pallas import tpu_sc as plsc

NC = 2
NS = 16
NW = NC * NS
LANES = 16
CHUNK = 80
NBUF = 4
GDEPTH = 2
IDEPTH = 3
NJ0 = 68
NJ1 = 188


def _bcast16(vec, e):
    idx = jnp.full((LANES, 1), e, jnp.int32)
    dn = lax.GatherDimensionNumbers(
        offset_dims=(), collapsed_slice_dims=(0,), start_index_map=(0,))
    return lax.gather(vec, idx, dn, slice_sizes=(1,),
                      mode=lax.GatherScatterMode.PROMISE_IN_BOUNDS)


def _sc_aggregate(x, packed, wpad):
    n, d = x.shape
    zrows = 80
    nblocks = n // zrows
    nslice = d // LANES

    mesh = plsc.VectorSubcoreMesh(core_axis_name="c", subcore_axis_name="s")
    cp = pltpu.CompilerParams()
    if "needs_layout_passes" in pltpu.CompilerParams.__dataclass_fields__:
        cp = dataclasses.replace(cp, needs_layout_passes=False)

    @functools.partial(
        pl.kernel,
        mesh=mesh,
        compiler_params=cp,
        out_type=jax.ShapeDtypeStruct((NC, n, d), jnp.float32),
        scratch_types=[
            pltpu.VMEM((NBUF, 2, CHUNK), jnp.int32),
            pltpu.VMEM((NBUF, 1, 128), jnp.float32),
            pltpu.VMEM((NBUF, CHUNK, d), jnp.float32),
            pltpu.VMEM_SHARED((n, d), jnp.float32),
            pltpu.SemaphoreType.DMA((NBUF,)),
            pltpu.SemaphoreType.DMA((NBUF,)),
            pltpu.SemaphoreType.DMA((NBUF,)),
        ],
    )
    def agg(x_hbm, pk_hbm, wk_hbm, out_hbm, pidx, wbuf, rows, acc,
            isem, gsem, ssem):
        cid = lax.axis_index("c")
        sid = lax.axis_index("s")
        njc = jnp.where(cid == 0, NJ0, NJ1)
        gbase = cid * NS * NJ0 + sid * njc

        zbuf = rows.at[0]
        def zrow(r, _):
            for f in range(nslice):
                zbuf.at[r, pl.ds(f * LANES, LANES)][...] = (
                    jnp.zeros((LANES,), jnp.float32))
            return 0
        lax.fori_loop(0, zrows, zrow, 0)
        nb = (nblocks - sid + NS - 1) // NS

        def zcopy(t, _):
            off = pl.multiple_of((sid + NS * t) * zrows, 8)
            pltpu.sync_copy(zbuf, acc.at[pl.ds(off, zrows)])
            return 0
        lax.fori_loop(0, nb, zcopy, 0)
        plsc.subcore_barrier()

        def idx_fire(c, bb):
            pltpu.async_copy(pk_hbm.at[gbase + c], pidx.at[bb], isem.at[bb])
            pltpu.async_copy(wk_hbm.at[gbase + c], wbuf.at[bb], isem.at[bb])

        def idx_wait(bb):
            pltpu.make_async_copy(
                pk_hbm.at[gbase], pidx.at[bb], isem.at[bb]).wait()
            pltpu.make_async_copy(
                wk_hbm.at[gbase], wbuf.at[bb], isem.at[bb]).wait()

        def gather_fire(bb):
            pltpu.async_copy(
                x_hbm.at[pidx.at[bb, 0]], rows.at[bb], gsem.at[bb])

        def gather_wait(bb):
            pltpu.make_async_copy(
                x_hbm.at[pidx.at[bb, 0]], rows.at[bb], gsem.at[bb]).wait()

        def scatter_fire(bb):
            pltpu.async_copy(
                rows.at[bb], acc.at[pidx.at[bb, 1]], ssem.at[bb], add=True)

        def scatter_wait(bb):
            pltpu.make_async_copy(
                rows.at[bb], acc.at[pidx.at[bb, 1]], ssem.at[bb]).wait()

        for b in range(IDEPTH):
            idx_fire(b, b)
        for b in range(GDEPTH):
            idx_wait(b)
            gather_fire(b)

        def tbody(t, _):
            for b in range(NBUF):
                c = t * NBUF + b
                gather_wait(b)

                rb = rows.at[b]

                def gbody(g, _):
                    goff = pl.multiple_of(g * LANES, LANES)
                    w16 = wbuf.at[b, 0, pl.ds(goff, LANES)][...]
                    for e in range(LANES):
                        b16 = _bcast16(w16, e)
                        r = goff + e
                        for f in range(nslice):
                            sl = rb.at[r, pl.ds(f * LANES, LANES)]
                            sl[...] = sl[...] * b16
                    return 0
                lax.fori_loop(0, CHUNK // LANES, gbody, 0)

                scatter_fire(b)

                bp = (b + IDEPTH) % NBUF

                @pl.when(c >= 1)
                def _():
                    scatter_wait(bp)

                @pl.when(c + IDEPTH < njc)
                def _():
                    idx_fire(c + IDEPTH, bp)

                bg = (b + GDEPTH) % NBUF

                @pl.when(c + GDEPTH < njc)
                def _():
                    idx_wait(bg)
                    gather_fire(bg)
            return 0
        lax.fori_loop(0, njc // NBUF, tbody, 0)

        scatter_wait(NBUF - 1)

        plsc.subcore_barrier()

        def ecopy(t, _):
            off = pl.multiple_of((sid + NS * t) * zrows, 8)
            pltpu.sync_copy(acc.at[pl.ds(off, zrows)],
                            out_hbm.at[cid].at[pl.ds(off, zrows)])
            return 0
        lax.fori_loop(0, nb, ecopy, 0)

    return agg(x, packed, wpad)


def _tc_combine(partials, W, b2):
    nc, n, d = partials.shape
    dout = W.shape[1]
    bm = 1000

    def mm(p_ref, w_ref, b_ref, o_ref):
        a = p_ref[0] + p_ref[1]
        o_ref[...] = (
            jnp.dot(a, w_ref[...], preferred_element_type=jnp.float32)
            + b_ref[...])

    return pl.pallas_call(
        mm,
        grid=(n // bm,),
        in_specs=[
            pl.BlockSpec((nc, bm, d), lambda i: (0, i, 0)),
            pl.BlockSpec((d, dout), lambda i: (0, 0)),
            pl.BlockSpec((1, dout), lambda i: (0, 0)),
        ],
        out_specs=pl.BlockSpec((bm, dout), lambda i: (i, 0)),
        out_shape=jax.ShapeDtypeStruct((n, dout), jnp.float32),
    )(partials, W, b2)


def kernel(input, edge_index, edge_weight, W, b):
    src = edge_index[1].astype(jnp.int32)
    dst = edge_index[0].astype(jnp.int32)
    ew = edge_weight.astype(jnp.float32)

    e = ew.shape[0]
    assert NJ0 % NBUF == 0 and NJ1 % NBUF == 0
    e_pad = NS * (NJ0 + NJ1) * CHUNK
    assert e_pad >= e
    nj = NJ0 + NJ1
    pad = e_pad - e
    src2 = jnp.concatenate([src, jnp.zeros((pad,), jnp.int32)])
    dst2 = jnp.concatenate([dst, jnp.zeros((pad,), jnp.int32)])
    packed = jnp.stack(
        [src2.reshape(NS * nj, CHUNK),
         dst2.reshape(NS * nj, CHUNK)], axis=1)
    w2 = jnp.concatenate([ew, jnp.zeros((pad,), jnp.float32)])
    wpad = jnp.pad(w2.reshape(NS * nj, CHUNK),
                   ((0, 0), (0, 128 - CHUNK))).reshape(NS * nj, 1, 128)

    partials = _sc_aggregate(input, packed, wpad)
    return _tc_combine(partials, W, b.reshape(1, -1))


# speedup vs baseline: 1.0405x; 1.0405x over previous
import dataclasses
import functools

import jax
import jax.numpy as jnp
from jax import lax
from jax.experimental import pallas as pl
from jax.experimental.pallas import tpu as pltpu
from jax.experimental.pallas import tpu_sc as plsc

NC = 2
NS = 16
NW = NC * NS
LANES = 16
CHUNK = 64
NBUF = 5
GDEPTH = 3
IDEPTH = 4
NJ0 = 160
NJ1 = 160


def _bcast16(vec, e):
    idx = jnp.full((LANES, 1), e, jnp.int32)
    dn = lax.GatherDimensionNumbers(
        offset_dims=(), collapsed_slice_dims=(0,), start_index_map=(0,))
    return lax.gather(vec, idx, dn, slice_sizes=(1,),
                      mode=lax.GatherScatterMode.PROMISE_IN_BOUNDS)


def _sc_aggregate(x, packed, wpad):
    n, d = x.shape
    zrows = 40
    nblocks = n // zrows
    nslice = d // LANES

    mesh = plsc.VectorSubcoreMesh(core_axis_name="c", subcore_axis_name="s")
    cp = pltpu.CompilerParams()
    if "needs_layout_passes" in pltpu.CompilerParams.__dataclass_fields__:
        cp = dataclasses.replace(cp, needs_layout_passes=False)

    @functools.partial(
        pl.kernel,
        mesh=mesh,
        compiler_params=cp,
        out_type=jax.ShapeDtypeStruct((NC, n, d), jnp.float32),
        scratch_types=[
            pltpu.VMEM((NBUF, 2, CHUNK), jnp.int32),
            pltpu.VMEM((NBUF, 1, 128), jnp.float32),
            pltpu.VMEM((NBUF, CHUNK, d), jnp.float32),
            pltpu.VMEM_SHARED((n, d), jnp.float32),
            pltpu.SemaphoreType.DMA((NBUF,)),
            pltpu.SemaphoreType.DMA((NBUF,)),
            pltpu.SemaphoreType.DMA((NBUF,)),
        ],
    )
    def agg(x_hbm, pk_hbm, wk_hbm, out_hbm, pidx, wbuf, rows, acc,
            isem, gsem, ssem):
        cid = lax.axis_index("c")
        sid = lax.axis_index("s")
        njc = jnp.where(cid == 0, NJ0, NJ1)
        gbase = cid * NS * NJ0 + sid * njc

        zbuf = rows.at[0].at[pl.ds(0, zrows)]
        def zrow(r, _):
            for f in range(nslice):
                zbuf.at[r, pl.ds(f * LANES, LANES)][...] = (
                    jnp.zeros((LANES,), jnp.float32))
            return 0
        lax.fori_loop(0, zrows, zrow, 0)
        nb = (nblocks - sid + NS - 1) // NS

        def zcopy(t, _):
            off = pl.multiple_of((sid + NS * t) * zrows, 8)
            pltpu.sync_copy(zbuf, acc.at[pl.ds(off, zrows)])
            return 0
        lax.fori_loop(0, nb, zcopy, 0)
        plsc.subcore_barrier()

        def idx_fire(c, bb):
            pltpu.async_copy(pk_hbm.at[gbase + c], pidx.at[bb], isem.at[bb])
            pltpu.async_copy(wk_hbm.at[gbase + c], wbuf.at[bb], isem.at[bb])

        def idx_wait(bb):
            pltpu.make_async_copy(
                pk_hbm.at[gbase], pidx.at[bb], isem.at[bb]).wait()
            pltpu.make_async_copy(
                wk_hbm.at[gbase], wbuf.at[bb], isem.at[bb]).wait()

        def gather_fire(bb):
            pltpu.async_copy(
                x_hbm.at[pidx.at[bb, 0]], rows.at[bb], gsem.at[bb])

        def gather_wait(bb):
            pltpu.make_async_copy(
                x_hbm.at[pidx.at[bb, 0]], rows.at[bb], gsem.at[bb]).wait()

        def scatter_fire(bb):
            pltpu.async_copy(
                rows.at[bb], acc.at[pidx.at[bb, 1]], ssem.at[bb], add=True)

        def scatter_wait(bb):
            pltpu.make_async_copy(
                rows.at[bb], acc.at[pidx.at[bb, 1]], ssem.at[bb]).wait()

        for b in range(IDEPTH):
            idx_fire(b, b)
        for b in range(GDEPTH):
            idx_wait(b)
            gather_fire(b)

        def tbody(t, _):
            for b in range(NBUF):
                c = t * NBUF + b
                gather_wait(b)

                rb = rows.at[b]

                def gbody(g, _):
                    goff = pl.multiple_of(g * LANES, LANES)
                    w16 = wbuf.at[b, 0, pl.ds(goff, LANES)][...]
                    for e in range(LANES):
                        b16 = _bcast16(w16, e)
                        r = goff + e
                        for f in range(nslice):
                            sl = rb.at[r, pl.ds(f * LANES, LANES)]
                            sl[...] = sl[...] * b16
                    return 0
                lax.fori_loop(0, CHUNK // LANES, gbody, 0)

                scatter_fire(b)

                bp = (b + IDEPTH) % NBUF

                @pl.when(c >= 1)
                def _():
                    scatter_wait(bp)

                @pl.when(c + IDEPTH < njc)
                def _():
                    idx_fire(c + IDEPTH, bp)

                bg = (b + GDEPTH) % NBUF

                @pl.when(c + GDEPTH < njc)
                def _():
                    idx_wait(bg)
                    gather_fire(bg)
            return 0
        lax.fori_loop(0, njc // NBUF, tbody, 0)

        scatter_wait(NBUF - 1)

        plsc.subcore_barrier()

        def ecopy(t, _):
            off = pl.multiple_of((sid + NS * t) * zrows, 8)
            pltpu.sync_copy(acc.at[pl.ds(off, zrows)],
                            out_hbm.at[cid].at[pl.ds(off, zrows)])
            return 0
        lax.fori_loop(0, nb, ecopy, 0)

    return agg(x, packed, wpad)


def _tc_combine(partials, W, b2):
    nc, n, d = partials.shape
    dout = W.shape[1]
    bm = 1000

    def mm(p_ref, w_ref, b_ref, o_ref):
        a = p_ref[0] + p_ref[1]
        o_ref[...] = (
            jnp.dot(a, w_ref[...], preferred_element_type=jnp.float32)
            + b_ref[...])

    return pl.pallas_call(
        mm,
        grid=(n // bm,),
        in_specs=[
            pl.BlockSpec((nc, bm, d), lambda i: (0, i, 0)),
            pl.BlockSpec((d, dout), lambda i: (0, 0)),
            pl.BlockSpec((1, dout), lambda i: (0, 0)),
        ],
        out_specs=pl.BlockSpec((bm, dout), lambda i: (i, 0)),
        out_shape=jax.ShapeDtypeStruct((n, dout), jnp.float32),
    )(partials, W, b2)


def kernel(input, edge_index, edge_weight, W, b):
    src = edge_index[1].astype(jnp.int32)
    dst = edge_index[0].astype(jnp.int32)
    ew = edge_weight.astype(jnp.float32)

    e = ew.shape[0]
    assert NJ0 % NBUF == 0 and NJ1 % NBUF == 0
    e_pad = NS * (NJ0 + NJ1) * CHUNK
    assert e_pad >= e
    nj = NJ0 + NJ1
    pad = e_pad - e
    src2 = jnp.concatenate([src, jnp.zeros((pad,), jnp.int32)])
    dst2 = jnp.concatenate([dst, jnp.zeros((pad,), jnp.int32)])
    packed = jnp.stack(
        [src2.reshape(NS * nj, CHUNK),
         dst2.reshape(NS * nj, CHUNK)], axis=1)
    w2 = jnp.concatenate([ew, jnp.zeros((pad,), jnp.float32)])
    wpad = jnp.pad(w2.reshape(NS * nj, CHUNK),
                   ((0, 0), (0, 128 - CHUNK))).reshape(NS * nj, 1, 128)

    partials = _sc_aggregate(input, packed, wpad)
    return _tc_combine(partials, W, b.reshape(1, -1))


# speedup vs baseline: 1.0790x; 1.0370x over previous
import dataclasses
import functools

import jax
import jax.numpy as jnp
from jax import lax
from jax.experimental import pallas as pl
from jax.experimental.pallas import tpu as pltpu
from jax.experimental.pallas import tpu_sc as plsc

NC = 2
NS = 16
NW = NC * NS
LANES = 16
CHUNK = 128


def _sc_aggregate(x, src, dst, w):
    n, d = x.shape
    e = w.shape[0]
    assert e % CHUNK == 0
    nchunks = e // CHUNK
    zrows = 80
    nblocks = n // zrows
    nslice = d // LANES

    mesh = plsc.VectorSubcoreMesh(core_axis_name="c", subcore_axis_name="s")
    cp = pltpu.CompilerParams()
    if "needs_layout_passes" in pltpu.CompilerParams.__dataclass_fields__:
        cp = dataclasses.replace(cp, needs_layout_passes=False)

    @functools.partial(
        pl.kernel,
        mesh=mesh,
        compiler_params=cp,
        out_type=jax.ShapeDtypeStruct((NC, n, d), jnp.float32),
        scratch_types=[
            pltpu.VMEM((CHUNK,), jnp.int32),
            pltpu.VMEM((CHUNK,), jnp.int32),
            pltpu.VMEM((CHUNK,), jnp.float32),
            pltpu.VMEM((CHUNK, d), jnp.float32),
            pltpu.VMEM((zrows, d), jnp.float32),
            pltpu.VMEM_SHARED((n, d), jnp.float32),
            pltpu.SemaphoreType.DMA,
        ],
    )
    def agg(x_hbm, src_hbm, dst_hbm, w_hbm, out_hbm,
            sidx, didx, wv, rows, zbuf, acc, sem):
        cid = lax.axis_index("c")
        sid = lax.axis_index("s")
        wid = sid * NC + cid

        def zrow(r, _):
            for f in range(nslice):
                zbuf.at[r, pl.ds(f * LANES, LANES)][...] = (
                    jnp.zeros((LANES,), jnp.float32))
            return 0
        lax.fori_loop(0, zrows, zrow, 0)
        nb = (nblocks - sid + NS - 1) // NS

        def zcopy(t, _):
            off = pl.multiple_of((sid + NS * t) * zrows, zrows)
            pltpu.sync_copy(zbuf, acc.at[pl.ds(off, zrows)])
            return 0
        lax.fori_loop(0, nb, zcopy, 0)
        plsc.subcore_barrier()

        nj = (nchunks - wid + NW - 1) // NW

        def chunk_body(j, _):
            base = pl.multiple_of((wid + NW * j) * CHUNK, CHUNK)
            pltpu.sync_copy(src_hbm.at[pl.ds(base, CHUNK)], sidx)
            pltpu.sync_copy(dst_hbm.at[pl.ds(base, CHUNK)], didx)
            pltpu.sync_copy(w_hbm.at[pl.ds(base, CHUNK)], wv)
            pltpu.async_copy(x_hbm.at[sidx], rows, sem).wait()

            def edge_body(r, _):
                bidx = jnp.full((LANES,), r, jnp.int32)
                b16 = plsc.load_gather(wv, [bidx])
                for f in range(nslice):
                    sl = rows.at[r, pl.ds(f * LANES, LANES)]
                    sl[...] = sl[...] * b16
                return 0
            lax.fori_loop(0, CHUNK, edge_body, 0)

            pltpu.sync_copy(rows, acc.at[didx], add=True)
            return 0
        lax.fori_loop(0, nj, chunk_body, 0)

        plsc.subcore_barrier()

        def ecopy(t, _):
            off = pl.multiple_of((sid + NS * t) * zrows, zrows)
            pltpu.sync_copy(acc.at[pl.ds(off, zrows)],
                            out_hbm.at[cid].at[pl.ds(off, zrows)])
            return 0
        lax.fori_loop(0, nb, ecopy, 0)

    return agg(x, src, dst, w)


def _tc_combine(partials, W, b2):
    nc, n, d = partials.shape
    dout = W.shape[1]
    bm = 1000

    def mm(p_ref, w_ref, b_ref, o_ref):
        a = p_ref[0] + p_ref[1]
        o_ref[...] = (
            jnp.dot(a, w_ref[...], preferred_element_type=jnp.float32)
            + b_ref[...])

    return pl.pallas_call(
        mm,
        grid=(n // bm,),
        in_specs=[
            pl.BlockSpec((nc, bm, d), lambda i: (0, i, 0)),
            pl.BlockSpec((d, dout), lambda i: (0, 0)),
            pl.BlockSpec((1, dout), lambda i: (0, 0)),
        ],
        out_specs=pl.BlockSpec((bm, dout), lambda i: (i, 0)),
        out_shape=jax.ShapeDtypeStruct((n, dout), jnp.float32),
    )(partials, W, b2)


def kernel(input, edge_index, edge_weight, W, b):
    src = edge_index[1].astype(jnp.int32)
    dst = edge_index[0].astype(jnp.int32)
    ew = edge_weight.astype(jnp.float32)
    partials = _sc_aggregate(input, src, dst, ew)
    return _tc_combine(partials, W, b.reshape(1, -1))
